# Initial kernel scaffold; baseline (speedup 1.0000x reference)
import jax
import jax.numpy as jnp
from jax.experimental import pallas as pl


def kernel(input, params):
    raise NotImplementedError("write your pallas kernel here")


# speedup vs baseline: 6.0056x; 6.0056x over previous
import functools

import jax
import jax.numpy as jnp
from jax import lax
from jax.experimental import pallas as pl
from jax.experimental.pallas import tpu as pltpu
from jax.experimental.pallas import tpu_sc as plsc

_EPSILON = 1e-6
_SIGMA_BOOST = 2.0
_IN_NUM = 16384
_OUT_NUM = 16384
_K = 8
_RADD = 4
_GADD = 4
_REGION = 128.0
_SIGMA_SCALE = 0.2
_MIN_SIGMA = 0.0
_D = 64
_N = _K * (2 + _RADD + _GADD)


def _make_indices(means):
    c = means.shape[0]
    key = jax.random.key(42)
    kr, kg = jax.random.split(key)
    n0 = jnp.floor(means)
    n1 = jnp.ceil(means)
    neighbor = jnp.stack([n0, n1], axis=2).astype(jnp.int32)
    u = jax.random.uniform(kr, (c, _K, _RADD, 1), dtype=jnp.float32) * (1.0 - _EPSILON)
    mns = jnp.round(means)[:, :, None, :]
    lower = mns - _REGION * 0.5
    upper = mns + _REGION * 0.5
    lower = jnp.where(lower < 0.0, 0.0, lower)
    lower = jnp.where(upper > float(_IN_NUM), float(_IN_NUM) - _REGION, lower)
    rr = (u * _REGION + lower).astype(jnp.int32)
    g = (jax.random.uniform(kg, (c, _K, _GADD, 1), dtype=jnp.float32)
         * (1.0 - _EPSILON) * _IN_NUM).astype(jnp.int32)
    ints = jnp.concatenate([neighbor, rr, g], axis=2)
    return ints.reshape(c, _N)


_CB = 64


def _weights_body(idx_ref, means_ref, sigmas_ref, values_ref, w_ref):
    u = idx_ref[...]
    uf = u.astype(jnp.float32)
    eq = u[:, :, None] == u[:, None, :]
    jj = lax.broadcasted_iota(jnp.int32, (_N, _N), 0)
    qq = lax.broadcasted_iota(jnp.int32, (_N, _N), 1)
    dup = jnp.any(eq & (qq < jj)[None, :, :], axis=2)

    m = means_ref[...]
    s = sigmas_ref[...]
    v = values_ref[...]
    ssq = jnp.sqrt(1.0 / (_EPSILON + s))
    d = (uf[:, :, None] - m[:, None, :]) * ssq[:, None, :]
    props = jnp.exp(-0.5 * d * d)
    props = jnp.where(dup[:, :, None], 0.0, props)
    denom = jnp.sum(props, axis=1, keepdims=True)
    props = props / denom
    w = jnp.sum(props * v[:, None, :], axis=2)
    c0 = pl.program_id(0) * _CB
    cids = c0 + lax.broadcasted_iota(jnp.int32, (_CB, _N), 0)
    w_ref[...] = jnp.where(u == cids, 0.0, w)


def _compute_weights(idx, means, sigmas, values):
    grid = (_OUT_NUM // _CB,)
    row_spec = lambda i: (i, 0)
    return pl.pallas_call(
        _weights_body,
        grid=grid,
        in_specs=[
            pl.BlockSpec((_CB, _N), row_spec),
            pl.BlockSpec((_CB, _K), row_spec),
            pl.BlockSpec((_CB, _K), row_spec),
            pl.BlockSpec((_CB, _K), row_spec),
        ],
        out_specs=pl.BlockSpec((_CB, _N), row_spec),
        out_shape=jax.ShapeDtypeStruct((_OUT_NUM, _N), jnp.float32),
    )(idx, means, sigmas, values)


_NC = 2
_NS = 16
_NW = _NC * _NS
_RPW = _OUT_NUM // _NW
_CHUNK = 32
_NCHUNK = _RPW // _CHUNK


def _sc_body(inp_hbm, idx_hbm, w_hbm, out_hbm, idx_v, w_v, rows_v, out_v, gsem):
    wid = lax.axis_index("s") * _NC + lax.axis_index("c")
    base = wid * _RPW

    def chunk_body(ci, _):
        row0 = base + ci * _CHUNK
        pltpu.sync_copy(idx_hbm.at[pl.ds(row0, _CHUNK)], idx_v)
        pltpu.sync_copy(w_hbm.at[pl.ds(row0, _CHUNK)], w_v)

        def row_body(r, _):
            pltpu.async_copy(inp_hbm.at[idx_v.at[r]], rows_v, gsem).wait()
            wvecs = [w_v[r, pl.ds(g * 16, 16)] for g in range(_N // 16)]
            accs = [jnp.zeros((16,), jnp.float32) for _ in range(_D // 16)]
            for j in range(_N):
                wj = wvecs[j // 16][j % 16]
                for q in range(_D // 16):
                    accs[q] = accs[q] + wj * rows_v[j, pl.ds(q * 16, 16)]
            for q in range(_D // 16):
                out_v[r, pl.ds(q * 16, 16)] = accs[q]
            return 0

        lax.fori_loop(0, _CHUNK, row_body, 0)
        pltpu.sync_copy(out_v, out_hbm.at[pl.ds(row0, _CHUNK)])
        return 0

    lax.fori_loop(0, _NCHUNK, chunk_body, 0)


def _sc_gather_reduce(inp, idx, w):
    mesh = plsc.VectorSubcoreMesh(core_axis_name="c", subcore_axis_name="s")
    run = functools.partial(
        pl.kernel,
        out_type=jax.ShapeDtypeStruct((_OUT_NUM, _D), jnp.float32),
        mesh=mesh,
        compiler_params=pltpu.CompilerParams(use_tc_tiling_on_sc=False),
        scratch_types=[
            pltpu.VMEM((_CHUNK, _N), jnp.int32),
            pltpu.VMEM((_CHUNK, _N), jnp.float32),
            pltpu.VMEM((_N, _D), jnp.float32),
            pltpu.VMEM((_CHUNK, _D), jnp.float32),
            pltpu.SemaphoreType.DMA,
        ],
    )(_sc_body)
    return run(inp, idx, w)


def kernel(input, params):
    means_f = jax.nn.sigmoid(params[:, 0:1]) * (_IN_NUM - 1)
    sigmas_f = jax.nn.softplus(params[:, 1:2] + _SIGMA_BOOST) + _EPSILON
    sigmas_f = sigmas_f * _IN_NUM * _SIGMA_SCALE + _MIN_SIGMA
    values = params[:, 2].reshape(_OUT_NUM, _K)
    means = means_f.reshape(_OUT_NUM, _K, 1)
    sigmas = sigmas_f.reshape(_OUT_NUM, _K)

    idx = _make_indices(lax.stop_gradient(means))
    w = _compute_weights(idx, means.reshape(_OUT_NUM, _K), sigmas, values)
    return _sc_gather_reduce(input, idx, w)


# speedup vs baseline: 12.3630x; 2.0586x over previous
import functools

import jax
import jax.numpy as jnp
from jax import lax
from jax.experimental import pallas as pl
from jax.experimental.pallas import tpu as pltpu
from jax.experimental.pallas import tpu_sc as plsc

_EPSILON = 1e-6
_SIGMA_BOOST = 2.0
_IN_NUM = 16384
_OUT_NUM = 16384
_K = 8
_RADD = 4
_GADD = 4
_REGION = 128.0
_SIGMA_SCALE = 0.2
_MIN_SIGMA = 0.0
_D = 64
_N = _K * (2 + _RADD + _GADD)


def _make_indices(means):
    c = means.shape[0]
    key = jax.random.key(42)
    kr, kg = jax.random.split(key)
    n0 = jnp.floor(means)
    n1 = jnp.ceil(means)
    neighbor = jnp.stack([n0, n1], axis=2).astype(jnp.int32)
    u = jax.random.uniform(kr, (c, _K, _RADD, 1), dtype=jnp.float32) * (1.0 - _EPSILON)
    mns = jnp.round(means)[:, :, None, :]
    lower = mns - _REGION * 0.5
    upper = mns + _REGION * 0.5
    lower = jnp.where(lower < 0.0, 0.0, lower)
    lower = jnp.where(upper > float(_IN_NUM), float(_IN_NUM) - _REGION, lower)
    rr = (u * _REGION + lower).astype(jnp.int32)
    g = (jax.random.uniform(kg, (c, _K, _GADD, 1), dtype=jnp.float32)
         * (1.0 - _EPSILON) * _IN_NUM).astype(jnp.int32)
    ints = jnp.concatenate([neighbor, rr, g], axis=2)
    return ints.reshape(c, _N)


_CB = 128


def _weights_body(idx_ref, means_ref, sigmas_ref, values_ref, w_ref):
    u = idx_ref[...]
    uf = u.astype(jnp.float32)
    eq = u[:, :, None] == u[:, None, :]
    multf = jnp.sum(eq.astype(jnp.float32), axis=1)
    inv_mult = 1.0 / multf

    m = means_ref[...]
    s = sigmas_ref[...]
    v = values_ref[...]
    ssq = jnp.sqrt(1.0 / (_EPSILON + s))
    d = (uf[:, None, :] - m[:, :, None]) * ssq[:, :, None]
    props = jnp.exp(-0.5 * d * d) * inv_mult[:, None, :]
    denom = jnp.sum(props, axis=2, keepdims=True)
    w = jnp.sum(props * (v[:, :, None] / denom), axis=1)
    c0 = pl.program_id(0) * _CB
    cids = c0 + lax.broadcasted_iota(jnp.int32, (_CB, _N), 0)
    w_ref[...] = jnp.where(u == cids, 0.0, w)


def _compute_weights(idx, means, sigmas, values):
    grid = (_OUT_NUM // _CB,)
    row_spec = lambda i: (i, 0)
    return pl.pallas_call(
        _weights_body,
        grid=grid,
        in_specs=[
            pl.BlockSpec((_CB, _N), row_spec),
            pl.BlockSpec((_CB, _K), row_spec),
            pl.BlockSpec((_CB, _K), row_spec),
            pl.BlockSpec((_CB, _K), row_spec),
        ],
        out_specs=pl.BlockSpec((_CB, _N), row_spec),
        out_shape=jax.ShapeDtypeStruct((_OUT_NUM, _N), jnp.float32),
    )(idx, means, sigmas, values)


_NC = 2
_NS = 16
_NW = _NC * _NS
_RPW = _OUT_NUM // _NW
_NBUF = 2


def _sc_body(inp_hbm, idx_hbm, w_hbm, out_hbm, idx_v, w_v, rows_v, out_v,
             sem0, sem1):
    wid = lax.axis_index("s") * _NC + lax.axis_index("c")
    base = wid * _RPW
    pltpu.sync_copy(idx_hbm.at[pl.ds(base, _RPW)], idx_v)
    pltpu.sync_copy(w_hbm.at[pl.ds(base, _RPW)], w_v)
    sems = (sem0, sem1)

    for b in range(_NBUF):
        pltpu.async_copy(inp_hbm.at[idx_v.at[b]], rows_v.at[b], sems[b])

    def pair_body(i, _):
        r0 = i * _NBUF
        for b in range(_NBUF):
            r = r0 + b
            pltpu.make_async_copy(
                inp_hbm.at[idx_v.at[r]], rows_v.at[b], sems[b]).wait()
            wvecs = [w_v[r, pl.ds(g * 16, 16)] for g in range(_N // 16)]
            accs = [jnp.zeros((16,), jnp.float32) for _ in range(_D // 16)]
            for j in range(_N):
                wj = wvecs[j // 16][j % 16]
                for q in range(_D // 16):
                    accs[q] = accs[q] + wj * rows_v[b, j, pl.ds(q * 16, 16)]
            for q in range(_D // 16):
                out_v[r, pl.ds(q * 16, 16)] = accs[q]

            @pl.when(r + _NBUF < _RPW)
            def _():
                pltpu.async_copy(
                    inp_hbm.at[idx_v.at[r + _NBUF]], rows_v.at[b], sems[b])
        return 0

    lax.fori_loop(0, _RPW // _NBUF, pair_body, 0)
    pltpu.sync_copy(out_v, out_hbm.at[pl.ds(base, _RPW)])


def _sc_gather_reduce(inp, idx, w):
    mesh = plsc.VectorSubcoreMesh(core_axis_name="c", subcore_axis_name="s")
    run = functools.partial(
        pl.kernel,
        out_type=jax.ShapeDtypeStruct((_OUT_NUM, _D), jnp.float32),
        mesh=mesh,
        compiler_params=pltpu.CompilerParams(use_tc_tiling_on_sc=False),
        scratch_types=[
            pltpu.VMEM((_RPW, _N), jnp.int32),
            pltpu.VMEM((_RPW, _N), jnp.float32),
            pltpu.VMEM((_NBUF, _N, _D), jnp.float32),
            pltpu.VMEM((_RPW, _D), jnp.float32),
            pltpu.SemaphoreType.DMA,
            pltpu.SemaphoreType.DMA,
        ],
    )(_sc_body)
    return run(inp, idx, w)


def kernel(input, params):
    means_f = jax.nn.sigmoid(params[:, 0:1]) * (_IN_NUM - 1)
    sigmas_f = jax.nn.softplus(params[:, 1:2] + _SIGMA_BOOST) + _EPSILON
    sigmas_f = sigmas_f * _IN_NUM * _SIGMA_SCALE + _MIN_SIGMA
    values = params[:, 2].reshape(_OUT_NUM, _K)
    means = means_f.reshape(_OUT_NUM, _K, 1)
    sigmas = sigmas_f.reshape(_OUT_NUM, _K)

    idx = _make_indices(lax.stop_gradient(means))
    w = _compute_weights(idx, means.reshape(_OUT_NUM, _K), sigmas, values)
    return _sc_gather_reduce(input, idx, w)


# speedup vs baseline: 16.1937x; 1.3099x over previous
import functools

import jax
import jax.numpy as jnp
import numpy as np
from jax import lax
from jax.experimental import pallas as pl
from jax.experimental.pallas import tpu as pltpu
from jax.experimental.pallas import tpu_sc as plsc

_EPSILON = 1e-6
_SIGMA_BOOST = 2.0
_IN_NUM = 16384
_OUT_NUM = 16384
_K = 8
_RADD = 4
_GADD = 4
_REGION = 128.0
_SIGMA_SCALE = 0.2
_MIN_SIGMA = 0.0
_D = 64
_N = _K * (2 + _RADD + _GADD)


_M32 = np.uint64(0xFFFFFFFF)


def _threefry2x32(k0, k1, x0, x1):
    x0 = x0.astype(np.uint64)
    x1 = x1.astype(np.uint64)
    ks = [np.uint64(k0) & _M32, np.uint64(k1) & _M32,
          (np.uint64(k0) ^ np.uint64(k1) ^ np.uint64(0x1BD11BDA)) & _M32]
    rot = [(13, 15, 26, 6), (17, 29, 16, 24)]
    x0 = (x0 + ks[0]) & _M32
    x1 = (x1 + ks[1]) & _M32
    sched = [(rot[0], ks[1], ks[2], 1), (rot[1], ks[2], ks[0], 2),
             (rot[0], ks[0], ks[1], 3), (rot[1], ks[1], ks[2], 4),
             (rot[0], ks[2], ks[0], 5)]
    for rs, a, b, i in sched:
        for r in rs:
            x0 = (x0 + x1) & _M32
            x1 = ((x1 << np.uint64(r)) | (x1 >> np.uint64(32 - r))) & _M32
            x1 = x0 ^ x1
        x0 = (x0 + a) & _M32
        x1 = (x1 + b + np.uint64(i)) & _M32
    return x0.astype(np.uint32), x1.astype(np.uint32)


def _np_uniform_01(partitionable, k, n):
    if partitionable:
        o0, o1 = _threefry2x32(k[0], k[1], np.zeros(n, np.uint32),
                               np.arange(n, dtype=np.uint32))
        bits = o0 ^ o1
    else:
        cnt = np.arange(n, dtype=np.uint32)
        h = n // 2
        o0, o1 = _threefry2x32(k[0], k[1], cnt[:h], cnt[h:])
        bits = np.concatenate([o0, o1])
    f = ((bits >> np.uint32(9)) | np.uint32(0x3F800000)).view(np.float32)
    return np.maximum(np.float32(0.0), f - np.float32(1.0))


@functools.lru_cache(maxsize=1)
def _uniform_consts():
    part = bool(jax.config.jax_threefry_partitionable)
    if part:
        o0, o1 = _threefry2x32(0, 42, np.zeros(2, np.uint32),
                               np.arange(2, dtype=np.uint32))
        kr = (int(o0[0]), int(o1[0]))
        kg = (int(o0[1]), int(o1[1]))
    else:
        o0, o1 = _threefry2x32(0, 42, np.arange(2, dtype=np.uint32),
                               np.arange(2, 4, dtype=np.uint32))
        kr = (int(o0[0]), int(o0[1]))
        kg = (int(o1[0]), int(o1[1]))
    n = _OUT_NUM * _K * _RADD
    u = _np_uniform_01(part, kr, n).reshape(_OUT_NUM, _K, _RADD, 1)
    u = u * np.float32(1.0 - _EPSILON)
    g = _np_uniform_01(part, kg, n).reshape(_OUT_NUM, _K, _GADD, 1)
    g = (g * np.float32(1.0 - _EPSILON) * np.float32(_IN_NUM)).astype(np.int32)
    return u, g


def _make_indices(means):
    c = means.shape[0]
    u, g = _uniform_consts()
    n0 = jnp.floor(means)
    n1 = jnp.ceil(means)
    neighbor = jnp.stack([n0, n1], axis=2).astype(jnp.int32)
    mns = jnp.round(means)[:, :, None, :]
    lower = mns - _REGION * 0.5
    upper = mns + _REGION * 0.5
    lower = jnp.where(lower < 0.0, 0.0, lower)
    lower = jnp.where(upper > float(_IN_NUM), float(_IN_NUM) - _REGION, lower)
    rr = (u * _REGION + lower).astype(jnp.int32)
    ints = jnp.concatenate([neighbor, rr, g], axis=2)
    return ints.reshape(c, _N)


_NC = 2
_NS = 16
_NW = _NC * _NS
_RPW = _OUT_NUM // _NW
_NBUF = 2
_L = 16


def _mesh():
    return plsc.VectorSubcoreMesh(core_axis_name="c", subcore_axis_name="s")


_DCHUNK = 128


def _dedup_body(idx_hbm, mask_hbm, idx_v, mask_v, table_v):
    wid = lax.axis_index("s") * _NC + lax.axis_index("c")
    base = wid * _RPW
    jconsts = [
        lax.iota(jnp.int32, _L) + g * _L for g in range(_N // _L)
    ]

    def chunk_body(ci, _):
        row0 = base + ci * _DCHUNK
        pltpu.sync_copy(idx_hbm.at[pl.ds(row0, _DCHUNK)], idx_v)

        def row_body(r, _):
            uvecs = [idx_v[r, pl.ds(g * _L, _L)] for g in range(_N // _L)]
            for g in range(_N // _L):
                plsc.store_scatter(table_v, [uvecs[g]], jconsts[g])
            for g in range(_N // _L):
                got = plsc.load_gather(table_v, [uvecs[g]])
                keep = jnp.where(got == jconsts[g], 1.0, 0.0)
                mask_v[r, pl.ds(g * _L, _L)] = keep
            return 0

        lax.fori_loop(0, _DCHUNK, row_body, 0)
        pltpu.sync_copy(mask_v, mask_hbm.at[pl.ds(row0, _DCHUNK)])
        return 0

    lax.fori_loop(0, _RPW // _DCHUNK, chunk_body, 0)


def _dedup_mask(idx):
    run = functools.partial(
        pl.kernel,
        out_type=jax.ShapeDtypeStruct((_OUT_NUM, _N), jnp.float32),
        mesh=_mesh(),
        compiler_params=pltpu.CompilerParams(needs_layout_passes=False),
        scratch_types=[
            pltpu.VMEM((_DCHUNK, _N), jnp.int32),
            pltpu.VMEM((_DCHUNK, _N), jnp.float32),
            pltpu.VMEM((_IN_NUM,), jnp.int32),
        ],
    )(_dedup_body)
    return run(idx)


_CB = 128


def _weights_body(idx_ref, mask_ref, means_ref, sigmas_ref, values_ref, w_ref):
    u = idx_ref[...]
    uf = u.astype(jnp.float32)
    keep = mask_ref[...]

    m = means_ref[...]
    s = sigmas_ref[...]
    v = values_ref[...]
    ssq = jnp.sqrt(1.0 / (_EPSILON + s))
    d = (uf[:, None, :] - m[:, :, None]) * ssq[:, :, None]
    props = jnp.exp(-0.5 * d * d) * keep[:, None, :]
    denom = jnp.sum(props, axis=2, keepdims=True)
    w = jnp.sum(props * (v[:, :, None] / denom), axis=1)
    c0 = pl.program_id(0) * _CB
    cids = c0 + lax.broadcasted_iota(jnp.int32, (_CB, _N), 0)
    w_ref[...] = jnp.where(u == cids, 0.0, w)


def _compute_weights(idx, mask, means, sigmas, values):
    grid = (_OUT_NUM // _CB,)
    row_spec = lambda i: (i, 0)
    return pl.pallas_call(
        _weights_body,
        grid=grid,
        in_specs=[
            pl.BlockSpec((_CB, _N), row_spec),
            pl.BlockSpec((_CB, _N), row_spec),
            pl.BlockSpec((_CB, _K), row_spec),
            pl.BlockSpec((_CB, _K), row_spec),
            pl.BlockSpec((_CB, _K), row_spec),
        ],
        out_specs=pl.BlockSpec((_CB, _N), row_spec),
        out_shape=jax.ShapeDtypeStruct((_OUT_NUM, _N), jnp.float32),
    )(idx, mask, means, sigmas, values)


def _sc_body(inp_hbm, idx_hbm, w_hbm, out_hbm, idx_v, w_v, rows_v, out_v,
             sem0, sem1):
    wid = lax.axis_index("s") * _NC + lax.axis_index("c")
    base = wid * _RPW
    pltpu.sync_copy(idx_hbm.at[pl.ds(base, _RPW)], idx_v)
    pltpu.sync_copy(w_hbm.at[pl.ds(base, _RPW)], w_v)
    sems = (sem0, sem1)

    for b in range(_NBUF):
        pltpu.async_copy(inp_hbm.at[idx_v.at[b]], rows_v.at[b], sems[b])

    def pair_body(i, _):
        r0 = i * _NBUF
        for b in range(_NBUF):
            r = r0 + b
            pltpu.make_async_copy(
                inp_hbm.at[idx_v.at[r]], rows_v.at[b], sems[b]).wait()
            wvecs = [w_v[r, pl.ds(g * _L, _L)] for g in range(_N // _L)]
            accs = [jnp.zeros((_L,), jnp.float32) for _ in range(_D // _L)]
            for j in range(_N):
                wj = wvecs[j // _L][j % _L]
                for q in range(_D // _L):
                    accs[q] = accs[q] + wj * rows_v[b, j, pl.ds(q * _L, _L)]
            for q in range(_D // _L):
                out_v[r, pl.ds(q * _L, _L)] = accs[q]

            @pl.when(r + _NBUF < _RPW)
            def _():
                pltpu.async_copy(
                    inp_hbm.at[idx_v.at[r + _NBUF]], rows_v.at[b], sems[b])
        return 0

    lax.fori_loop(0, _RPW // _NBUF, pair_body, 0)
    pltpu.sync_copy(out_v, out_hbm.at[pl.ds(base, _RPW)])


def _sc_gather_reduce(inp, idx, w):
    run = functools.partial(
        pl.kernel,
        out_type=jax.ShapeDtypeStruct((_OUT_NUM, _D), jnp.float32),
        mesh=_mesh(),
        compiler_params=pltpu.CompilerParams(use_tc_tiling_on_sc=False),
        scratch_types=[
            pltpu.VMEM((_RPW, _N), jnp.int32),
            pltpu.VMEM((_RPW, _N), jnp.float32),
            pltpu.VMEM((_NBUF, _N, _D), jnp.float32),
            pltpu.VMEM((_RPW, _D), jnp.float32),
            pltpu.SemaphoreType.DMA,
            pltpu.SemaphoreType.DMA,
        ],
    )(_sc_body)
    return run(inp, idx, w)


def kernel(input, params):
    means_f = jax.nn.sigmoid(params[:, 0:1]) * (_IN_NUM - 1)
    sigmas_f = jax.nn.softplus(params[:, 1:2] + _SIGMA_BOOST) + _EPSILON
    sigmas_f = sigmas_f * _IN_NUM * _SIGMA_SCALE + _MIN_SIGMA
    values = params[:, 2].reshape(_OUT_NUM, _K)
    means = means_f.reshape(_OUT_NUM, _K, 1)
    sigmas = sigmas_f.reshape(_OUT_NUM, _K)

    idx = _make_indices(lax.stop_gradient(means))
    mask = _dedup_mask(idx)
    w = _compute_weights(idx, mask, means.reshape(_OUT_NUM, _K), sigmas, values)
    return _sc_gather_reduce(input, idx, w)


# speedup vs baseline: 17.7729x; 1.0975x over previous
import functools

import jax
import jax.numpy as jnp
import numpy as np
from jax import lax
from jax.experimental import pallas as pl
from jax.experimental.pallas import tpu as pltpu
from jax.experimental.pallas import tpu_sc as plsc

_EPSILON = 1e-6
_SIGMA_BOOST = 2.0
_IN_NUM = 16384
_OUT_NUM = 16384
_K = 8
_RADD = 4
_GADD = 4
_REGION = 128.0
_SIGMA_SCALE = 0.2
_MIN_SIGMA = 0.0
_D = 64
_N = _K * (2 + _RADD + _GADD)


_M32 = np.uint64(0xFFFFFFFF)


def _threefry2x32(k0, k1, x0, x1):
    x0 = x0.astype(np.uint64)
    x1 = x1.astype(np.uint64)
    ks = [np.uint64(k0) & _M32, np.uint64(k1) & _M32,
          (np.uint64(k0) ^ np.uint64(k1) ^ np.uint64(0x1BD11BDA)) & _M32]
    rot = [(13, 15, 26, 6), (17, 29, 16, 24)]
    x0 = (x0 + ks[0]) & _M32
    x1 = (x1 + ks[1]) & _M32
    sched = [(rot[0], ks[1], ks[2], 1), (rot[1], ks[2], ks[0], 2),
             (rot[0], ks[0], ks[1], 3), (rot[1], ks[1], ks[2], 4),
             (rot[0], ks[2], ks[0], 5)]
    for rs, a, b, i in sched:
        for r in rs:
            x0 = (x0 + x1) & _M32
            x1 = ((x1 << np.uint64(r)) | (x1 >> np.uint64(32 - r))) & _M32
            x1 = x0 ^ x1
        x0 = (x0 + a) & _M32
        x1 = (x1 + b + np.uint64(i)) & _M32
    return x0.astype(np.uint32), x1.astype(np.uint32)


def _np_uniform_01(partitionable, k, n):
    if partitionable:
        o0, o1 = _threefry2x32(k[0], k[1], np.zeros(n, np.uint32),
                               np.arange(n, dtype=np.uint32))
        bits = o0 ^ o1
    else:
        cnt = np.arange(n, dtype=np.uint32)
        h = n // 2
        o0, o1 = _threefry2x32(k[0], k[1], cnt[:h], cnt[h:])
        bits = np.concatenate([o0, o1])
    f = ((bits >> np.uint32(9)) | np.uint32(0x3F800000)).view(np.float32)
    return np.maximum(np.float32(0.0), f - np.float32(1.0))


@functools.lru_cache(maxsize=1)
def _uniform_consts():
    part = bool(jax.config.jax_threefry_partitionable)
    if part:
        o0, o1 = _threefry2x32(0, 42, np.zeros(2, np.uint32),
                               np.arange(2, dtype=np.uint32))
        kr = (int(o0[0]), int(o1[0]))
        kg = (int(o0[1]), int(o1[1]))
    else:
        o0, o1 = _threefry2x32(0, 42, np.arange(2, dtype=np.uint32),
                               np.arange(2, 4, dtype=np.uint32))
        kr = (int(o0[0]), int(o0[1]))
        kg = (int(o1[0]), int(o1[1]))
    n = _OUT_NUM * _K * _RADD
    u = _np_uniform_01(part, kr, n).reshape(_OUT_NUM, _K, _RADD)
    u = u * np.float32(1.0 - _EPSILON) * np.float32(_REGION)
    g = _np_uniform_01(part, kg, n).reshape(_OUT_NUM, _K, _GADD)
    g = (g * np.float32(1.0 - _EPSILON) * np.float32(_IN_NUM)).astype(np.int32)
    ur = np.ascontiguousarray(u.transpose(0, 2, 1)).reshape(_OUT_NUM, _RADD * _K)
    gc = np.ascontiguousarray(g.transpose(0, 2, 1)).reshape(_OUT_NUM, _GADD * _K)
    return ur, gc


_CBA = 512


def _indices_body(means_ref, ur_ref, gc_ref, idx_ref):
    m = means_ref[...]
    idx_ref[:, pl.ds(0, _K)] = jnp.floor(m).astype(jnp.int32)
    idx_ref[:, pl.ds(_K, _K)] = jnp.ceil(m).astype(jnp.int32)
    mns = jnp.round(m)
    lower = mns - _REGION * 0.5
    upper = mns + _REGION * 0.5
    lower = jnp.where(lower < 0.0, 0.0, lower)
    lower = jnp.where(upper > float(_IN_NUM), float(_IN_NUM) - _REGION, lower)
    for s in range(_RADD):
        rr = (ur_ref[:, pl.ds(s * _K, _K)] + lower).astype(jnp.int32)
        idx_ref[:, pl.ds((2 + s) * _K, _K)] = rr
    for s in range(_GADD):
        idx_ref[:, pl.ds((2 + _RADD + s) * _K, _K)] = gc_ref[:, pl.ds(s * _K, _K)]


def _make_indices(means):
    ur, gc = _uniform_consts()
    grid = (_OUT_NUM // _CBA,)
    row_spec = lambda i: (i, 0)
    return pl.pallas_call(
        _indices_body,
        grid=grid,
        in_specs=[
            pl.BlockSpec((_CBA, _K), row_spec),
            pl.BlockSpec((_CBA, _RADD * _K), row_spec),
            pl.BlockSpec((_CBA, _GADD * _K), row_spec),
        ],
        out_specs=pl.BlockSpec((_CBA, _N), row_spec),
        out_shape=jax.ShapeDtypeStruct((_OUT_NUM, _N), jnp.int32),
    )(means, jnp.asarray(ur), jnp.asarray(gc))


_NC = 2
_NS = 16
_NW = _NC * _NS
_RPW = _OUT_NUM // _NW
_NBUF = 4
_L = 16


def _mesh():
    return plsc.VectorSubcoreMesh(core_axis_name="c", subcore_axis_name="s")


_DCHUNK = 128


def _dedup_body(idx_hbm, mask_hbm, idx_v, mask_v, table_v):
    wid = lax.axis_index("s") * _NC + lax.axis_index("c")
    base = wid * _RPW
    jconsts = [
        lax.iota(jnp.int32, _L) + g * _L for g in range(_N // _L)
    ]

    def chunk_body(ci, _):
        row0 = base + ci * _DCHUNK
        pltpu.sync_copy(idx_hbm.at[pl.ds(row0, _DCHUNK)], idx_v)

        def row_body(r, _):
            uvecs = [idx_v[r, pl.ds(g * _L, _L)] for g in range(_N // _L)]
            for g in range(_N // _L):
                plsc.store_scatter(table_v, [uvecs[g]], jconsts[g])
            for g in range(_N // _L):
                got = plsc.load_gather(table_v, [uvecs[g]])
                keep = jnp.where(got == jconsts[g], 1.0, 0.0)
                mask_v[r, pl.ds(g * _L, _L)] = keep
            return 0

        lax.fori_loop(0, _DCHUNK, row_body, 0)
        pltpu.sync_copy(mask_v, mask_hbm.at[pl.ds(row0, _DCHUNK)])
        return 0

    lax.fori_loop(0, _RPW // _DCHUNK, chunk_body, 0)


def _dedup_mask(idx):
    run = functools.partial(
        pl.kernel,
        out_type=jax.ShapeDtypeStruct((_OUT_NUM, _N), jnp.float32),
        mesh=_mesh(),
        compiler_params=pltpu.CompilerParams(needs_layout_passes=False),
        scratch_types=[
            pltpu.VMEM((_DCHUNK, _N), jnp.int32),
            pltpu.VMEM((_DCHUNK, _N), jnp.float32),
            pltpu.VMEM((_IN_NUM,), jnp.int32),
        ],
    )(_dedup_body)
    return run(idx)


_CB = 128


def _weights_body(idx_ref, mask_ref, means_ref, sigmas_ref, values_ref, w_ref):
    u = idx_ref[...]
    uf = u.astype(jnp.float32)
    keep = mask_ref[...]

    m = means_ref[...]
    s = sigmas_ref[...]
    v = values_ref[...]
    ssq = jnp.sqrt(1.0 / (_EPSILON + s))
    d = (uf[:, None, :] - m[:, :, None]) * ssq[:, :, None]
    props = jnp.exp(-0.5 * d * d) * keep[:, None, :]
    denom = jnp.sum(props, axis=2, keepdims=True)
    w = jnp.sum(props * (v[:, :, None] / denom), axis=1)
    c0 = pl.program_id(0) * _CB
    cids = c0 + lax.broadcasted_iota(jnp.int32, (_CB, _N), 0)
    w_ref[...] = jnp.where(u == cids, 0.0, w)


def _compute_weights(idx, mask, means, sigmas, values):
    grid = (_OUT_NUM // _CB,)
    row_spec = lambda i: (i, 0)
    return pl.pallas_call(
        _weights_body,
        grid=grid,
        in_specs=[
            pl.BlockSpec((_CB, _N), row_spec),
            pl.BlockSpec((_CB, _N), row_spec),
            pl.BlockSpec((_CB, _K), row_spec),
            pl.BlockSpec((_CB, _K), row_spec),
            pl.BlockSpec((_CB, _K), row_spec),
        ],
        out_specs=pl.BlockSpec((_CB, _N), row_spec),
        out_shape=jax.ShapeDtypeStruct((_OUT_NUM, _N), jnp.float32),
    )(idx, mask, means, sigmas, values)


_OCHUNK = 128


def _sc_body(inp_hbm, idx_hbm, w_hbm, out_hbm, idx_v, w_v, rows_v, out_v,
             *sems):
    wid = lax.axis_index("s") * _NC + lax.axis_index("c")
    base = wid * _RPW
    pltpu.sync_copy(idx_hbm.at[pl.ds(base, _RPW)], idx_v)
    pltpu.sync_copy(w_hbm.at[pl.ds(base, _RPW)], w_v)

    for b in range(_NBUF):
        pltpu.async_copy(inp_hbm.at[idx_v.at[b]], rows_v.at[b], sems[b])

    def group_body(i, _):
        r0 = i * _NBUF
        for b in range(_NBUF):
            r = r0 + b
            pltpu.make_async_copy(
                inp_hbm.at[idx_v.at[r]], rows_v.at[b], sems[b]).wait()
            wvecs = [w_v[r, pl.ds(g * _L, _L)] for g in range(_N // _L)]
            accs = [jnp.zeros((_L,), jnp.float32) for _ in range(_D // _L)]
            for j in range(_N):
                wj = wvecs[j // _L][j % _L]
                for q in range(_D // _L):
                    accs[q] = accs[q] + wj * rows_v[b, j, pl.ds(q * _L, _L)]
            ro = r & (_OCHUNK - 1)
            for q in range(_D // _L):
                out_v[ro, pl.ds(q * _L, _L)] = accs[q]

            @pl.when(r + _NBUF < _RPW)
            def _():
                pltpu.async_copy(
                    inp_hbm.at[idx_v.at[r + _NBUF]], rows_v.at[b], sems[b])

            @pl.when(ro == _OCHUNK - 1)
            def _():
                pltpu.sync_copy(
                    out_v, out_hbm.at[pl.ds(base + r - (_OCHUNK - 1), _OCHUNK)])
        return 0

    lax.fori_loop(0, _RPW // _NBUF, group_body, 0)


def _sc_gather_reduce(inp, idx, w):
    run = functools.partial(
        pl.kernel,
        out_type=jax.ShapeDtypeStruct((_OUT_NUM, _D), jnp.float32),
        mesh=_mesh(),
        compiler_params=pltpu.CompilerParams(use_tc_tiling_on_sc=False),
        scratch_types=[
            pltpu.VMEM((_RPW, _N), jnp.int32),
            pltpu.VMEM((_RPW, _N), jnp.float32),
            pltpu.VMEM((_NBUF, _N, _D), jnp.float32),
            pltpu.VMEM((_OCHUNK, _D), jnp.float32),
        ] + [pltpu.SemaphoreType.DMA] * _NBUF,
    )(_sc_body)
    return run(inp, idx, w)


def kernel(input, params):
    means_f = jax.nn.sigmoid(params[:, 0:1]) * (_IN_NUM - 1)
    sigmas_f = jax.nn.softplus(params[:, 1:2] + _SIGMA_BOOST) + _EPSILON
    sigmas_f = sigmas_f * _IN_NUM * _SIGMA_SCALE + _MIN_SIGMA
    values = params[:, 2].reshape(_OUT_NUM, _K)
    means = means_f.reshape(_OUT_NUM, _K, 1)
    sigmas = sigmas_f.reshape(_OUT_NUM, _K)

    means2 = means.reshape(_OUT_NUM, _K)
    idx = _make_indices(means2)
    mask = _dedup_mask(idx)
    w = _compute_weights(idx, mask, means2, sigmas, values)
    return _sc_gather_reduce(input, idx, w)


# speedup vs baseline: 18.6613x; 1.0500x over previous
import functools

import jax
import jax.numpy as jnp
import numpy as np
from jax import lax
from jax.experimental import pallas as pl
from jax.experimental.pallas import tpu as pltpu
from jax.experimental.pallas import tpu_sc as plsc

_EPSILON = 1e-6
_SIGMA_BOOST = 2.0
_IN_NUM = 16384
_OUT_NUM = 16384
_K = 8
_RADD = 4
_GADD = 4
_REGION = 128.0
_SIGMA_SCALE = 0.2
_MIN_SIGMA = 0.0
_D = 64
_N = _K * (2 + _RADD + _GADD)


_M32 = np.uint64(0xFFFFFFFF)


def _threefry2x32(k0, k1, x0, x1):
    x0 = x0.astype(np.uint64)
    x1 = x1.astype(np.uint64)
    ks = [np.uint64(k0) & _M32, np.uint64(k1) & _M32,
          (np.uint64(k0) ^ np.uint64(k1) ^ np.uint64(0x1BD11BDA)) & _M32]
    rot = [(13, 15, 26, 6), (17, 29, 16, 24)]
    x0 = (x0 + ks[0]) & _M32
    x1 = (x1 + ks[1]) & _M32
    sched = [(rot[0], ks[1], ks[2], 1), (rot[1], ks[2], ks[0], 2),
             (rot[0], ks[0], ks[1], 3), (rot[1], ks[1], ks[2], 4),
             (rot[0], ks[2], ks[0], 5)]
    for rs, a, b, i in sched:
        for r in rs:
            x0 = (x0 + x1) & _M32
            x1 = ((x1 << np.uint64(r)) | (x1 >> np.uint64(32 - r))) & _M32
            x1 = x0 ^ x1
        x0 = (x0 + a) & _M32
        x1 = (x1 + b + np.uint64(i)) & _M32
    return x0.astype(np.uint32), x1.astype(np.uint32)


def _np_uniform_01(partitionable, k, n):
    if partitionable:
        o0, o1 = _threefry2x32(k[0], k[1], np.zeros(n, np.uint32),
                               np.arange(n, dtype=np.uint32))
        bits = o0 ^ o1
    else:
        cnt = np.arange(n, dtype=np.uint32)
        h = n // 2
        o0, o1 = _threefry2x32(k[0], k[1], cnt[:h], cnt[h:])
        bits = np.concatenate([o0, o1])
    f = ((bits >> np.uint32(9)) | np.uint32(0x3F800000)).view(np.float32)
    return np.maximum(np.float32(0.0), f - np.float32(1.0))


@functools.lru_cache(maxsize=1)
def _uniform_consts():
    part = bool(jax.config.jax_threefry_partitionable)
    if part:
        o0, o1 = _threefry2x32(0, 42, np.zeros(2, np.uint32),
                               np.arange(2, dtype=np.uint32))
        kr = (int(o0[0]), int(o1[0]))
        kg = (int(o0[1]), int(o1[1]))
    else:
        o0, o1 = _threefry2x32(0, 42, np.arange(2, dtype=np.uint32),
                               np.arange(2, 4, dtype=np.uint32))
        kr = (int(o0[0]), int(o0[1]))
        kg = (int(o1[0]), int(o1[1]))
    n = _OUT_NUM * _K * _RADD
    u = _np_uniform_01(part, kr, n).reshape(_OUT_NUM, _K, _RADD)
    u = u * np.float32(1.0 - _EPSILON) * np.float32(_REGION)
    g = _np_uniform_01(part, kg, n).reshape(_OUT_NUM, _K, _GADD)
    g = (g * np.float32(1.0 - _EPSILON) * np.float32(_IN_NUM)).astype(np.int32)
    ur = np.ascontiguousarray(u.transpose(0, 2, 1)).reshape(_OUT_NUM, _RADD * _K)
    gc = np.ascontiguousarray(g.transpose(0, 2, 1)).reshape(_OUT_NUM, _GADD * _K)
    return ur, gc


_CBA = 512


def _indices_body(means_ref, ur_ref, gc_ref, idx_ref):
    m = means_ref[...]
    idx_ref[:, pl.ds(0, _K)] = jnp.floor(m).astype(jnp.int32)
    idx_ref[:, pl.ds(_K, _K)] = jnp.ceil(m).astype(jnp.int32)
    mns = jnp.round(m)
    lower = mns - _REGION * 0.5
    upper = mns + _REGION * 0.5
    lower = jnp.where(lower < 0.0, 0.0, lower)
    lower = jnp.where(upper > float(_IN_NUM), float(_IN_NUM) - _REGION, lower)
    for s in range(_RADD):
        rr = (ur_ref[:, pl.ds(s * _K, _K)] + lower).astype(jnp.int32)
        idx_ref[:, pl.ds((2 + s) * _K, _K)] = rr
    for s in range(_GADD):
        idx_ref[:, pl.ds((2 + _RADD + s) * _K, _K)] = gc_ref[:, pl.ds(s * _K, _K)]


def _make_indices(means):
    ur, gc = _uniform_consts()
    grid = (_OUT_NUM // _CBA,)
    row_spec = lambda i: (i, 0)
    return pl.pallas_call(
        _indices_body,
        grid=grid,
        in_specs=[
            pl.BlockSpec((_CBA, _K), row_spec),
            pl.BlockSpec((_CBA, _RADD * _K), row_spec),
            pl.BlockSpec((_CBA, _GADD * _K), row_spec),
        ],
        out_specs=pl.BlockSpec((_CBA, _N), row_spec),
        out_shape=jax.ShapeDtypeStruct((_OUT_NUM, _N), jnp.int32),
    )(means, jnp.asarray(ur), jnp.asarray(gc))


_NC = 2
_NS = 16
_NW = _NC * _NS
_RPW = _OUT_NUM // _NW
_NBUF = 4
_L = 16


def _mesh():
    return plsc.VectorSubcoreMesh(core_axis_name="c", subcore_axis_name="s")


_DCHUNK = 128


def _dedup_body(idx_hbm, mask_hbm, idx_v, mask_v, table_v):
    wid = lax.axis_index("s") * _NC + lax.axis_index("c")
    base = wid * _RPW
    jconsts = [
        lax.iota(jnp.int32, _L) + g * _L for g in range(_N // _L)
    ]

    def chunk_body(ci, _):
        row0 = base + ci * _DCHUNK
        pltpu.sync_copy(idx_hbm.at[pl.ds(row0, _DCHUNK)], idx_v)

        def row_body(r, _):
            uvecs = [idx_v[r, pl.ds(g * _L, _L)] for g in range(_N // _L)]
            for g in range(_N // _L):
                plsc.store_scatter(table_v, [uvecs[g]], jconsts[g])
            for g in range(_N // _L):
                got = plsc.load_gather(table_v, [uvecs[g]])
                keep = jnp.where(got == jconsts[g], 1.0, 0.0)
                mask_v[r, pl.ds(g * _L, _L)] = keep
            return 0

        lax.fori_loop(0, _DCHUNK, row_body, 0)
        pltpu.sync_copy(mask_v, mask_hbm.at[pl.ds(row0, _DCHUNK)])
        return 0

    lax.fori_loop(0, _RPW // _DCHUNK, chunk_body, 0)


def _dedup_mask(idx):
    run = functools.partial(
        pl.kernel,
        out_type=jax.ShapeDtypeStruct((_OUT_NUM, _N), jnp.float32),
        mesh=_mesh(),
        compiler_params=pltpu.CompilerParams(needs_layout_passes=False),
        scratch_types=[
            pltpu.VMEM((_DCHUNK, _N), jnp.int32),
            pltpu.VMEM((_DCHUNK, _N), jnp.float32),
            pltpu.VMEM((_IN_NUM,), jnp.int32),
        ],
    )(_dedup_body)
    return run(idx)


_CB = 128


def _weights_body(base, idx_ref, mask_ref, means_ref, sigmas_ref, values_ref,
                  w_ref):
    u = idx_ref[...]
    uf = u.astype(jnp.float32)
    keep = mask_ref[...]

    m = means_ref[...]
    s = sigmas_ref[...]
    v = values_ref[...]
    ssq = jnp.sqrt(1.0 / (_EPSILON + s))
    d = (uf[:, None, :] - m[:, :, None]) * ssq[:, :, None]
    props = jnp.exp(-0.5 * d * d) * keep[:, None, :]
    denom = jnp.sum(props, axis=2, keepdims=True)
    w = jnp.sum(props * (v[:, :, None] / denom), axis=1)
    c0 = base + pl.program_id(0) * _CB
    cids = c0 + lax.broadcasted_iota(jnp.int32, (_CB, _N), 0)
    w_ref[...] = jnp.where(u == cids, 0.0, w)


def _compute_weights(idx, mask, means, sigmas, values, base):
    rows = idx.shape[0]
    grid = (rows // _CB,)
    row_spec = lambda i: (i, 0)
    return pl.pallas_call(
        functools.partial(_weights_body, base),
        grid=grid,
        in_specs=[
            pl.BlockSpec((_CB, _N), row_spec),
            pl.BlockSpec((_CB, _N), row_spec),
            pl.BlockSpec((_CB, _K), row_spec),
            pl.BlockSpec((_CB, _K), row_spec),
            pl.BlockSpec((_CB, _K), row_spec),
        ],
        out_specs=pl.BlockSpec((_CB, _N), row_spec),
        out_shape=jax.ShapeDtypeStruct((rows, _N), jnp.float32),
    )(idx, mask, means, sigmas, values)


_OCHUNK = 128


def _sc_body(rpw, inp_hbm, idx_hbm, w_hbm, out_hbm, idx_v, w_v, rows_v, out_v,
             *sems):
    wid = lax.axis_index("s") * _NC + lax.axis_index("c")
    base = wid * rpw
    pltpu.sync_copy(idx_hbm.at[pl.ds(base, rpw)], idx_v)
    pltpu.sync_copy(w_hbm.at[pl.ds(base, rpw)], w_v)

    for b in range(_NBUF):
        pltpu.async_copy(inp_hbm.at[idx_v.at[b]], rows_v.at[b], sems[b])

    def group_body(i, _):
        r0 = i * _NBUF
        for b in range(_NBUF):
            r = r0 + b
            pltpu.make_async_copy(
                inp_hbm.at[idx_v.at[r]], rows_v.at[b], sems[b]).wait()
            wvecs = [w_v[r, pl.ds(g * _L, _L)] for g in range(_N // _L)]
            accs = [jnp.zeros((_L,), jnp.float32) for _ in range(_D // _L)]
            for j in range(_N):
                wj = wvecs[j // _L][j % _L]
                for q in range(_D // _L):
                    accs[q] = accs[q] + wj * rows_v[b, j, pl.ds(q * _L, _L)]
            ro = r & (_OCHUNK - 1)
            for q in range(_D // _L):
                out_v[ro, pl.ds(q * _L, _L)] = accs[q]

            @pl.when(r + _NBUF < rpw)
            def _():
                pltpu.async_copy(
                    inp_hbm.at[idx_v.at[r + _NBUF]], rows_v.at[b], sems[b])

            @pl.when(ro == _OCHUNK - 1)
            def _():
                pltpu.sync_copy(
                    out_v, out_hbm.at[pl.ds(base + r - (_OCHUNK - 1), _OCHUNK)])
        return 0

    lax.fori_loop(0, rpw // _NBUF, group_body, 0)


def _sc_gather_reduce(inp, idx, w):
    rows = idx.shape[0]
    rpw = rows // _NW
    run = functools.partial(
        pl.kernel,
        out_type=jax.ShapeDtypeStruct((rows, _D), jnp.float32),
        mesh=_mesh(),
        compiler_params=pltpu.CompilerParams(use_tc_tiling_on_sc=False),
        scratch_types=[
            pltpu.VMEM((rpw, _N), jnp.int32),
            pltpu.VMEM((rpw, _N), jnp.float32),
            pltpu.VMEM((_NBUF, _N, _D), jnp.float32),
            pltpu.VMEM((_OCHUNK, _D), jnp.float32),
        ] + [pltpu.SemaphoreType.DMA] * _NBUF,
    )(functools.partial(_sc_body, rpw))
    return run(inp, idx, w)


def kernel(input, params):
    means_f = jax.nn.sigmoid(params[:, 0:1]) * (_IN_NUM - 1)
    sigmas_f = jax.nn.softplus(params[:, 1:2] + _SIGMA_BOOST) + _EPSILON
    sigmas_f = sigmas_f * _IN_NUM * _SIGMA_SCALE + _MIN_SIGMA
    values = params[:, 2].reshape(_OUT_NUM, _K)
    means = means_f.reshape(_OUT_NUM, _K, 1)
    sigmas = sigmas_f.reshape(_OUT_NUM, _K)

    means2 = means.reshape(_OUT_NUM, _K)
    idx = _make_indices(means2)
    mask = _dedup_mask(idx)
    h = _OUT_NUM // 2
    outs = []
    for hi in range(2):
        sl = slice(hi * h, (hi + 1) * h)
        w_h = _compute_weights(idx[sl], mask[sl], means2[sl], sigmas[sl],
                               values[sl], hi * h)
        outs.append(_sc_gather_reduce(input, idx[sl], w_h))
    return jnp.concatenate(outs, axis=0)
